# baseline (speedup 1.0000x reference)
import numpy as np
import jax
import jax.numpy as jnp
from jax import lax
from jax.experimental import pallas as pl
from jax.experimental.pallas import tpu as pltpu

_STEPS = 200
_T1 = _STEPS + 1
_MAXFREQ = 100.0
_NITER = 24
_BN_A = 4096
_BN_B = 2048


def _tf_py(k0, k1, xs):
    MASK = 0xFFFFFFFF
    ks = [k0, k1, k0 ^ k1 ^ 0x1BD11BDA]
    rots = ((13, 15, 26, 6), (17, 29, 16, 24)) * 3
    out = []
    for x0, x1 in xs:
        x0 = (x0 + ks[0]) & MASK
        x1 = (x1 + ks[1]) & MASK
        for i in range(5):
            for r in rots[i]:
                x0 = (x0 + x1) & MASK
                x1 = ((x1 << r) | (x1 >> (32 - r))) & MASK
                x1 = x0 ^ x1
            x0 = (x0 + ks[(i + 1) % 3]) & MASK
            x1 = (x1 + ks[(i + 2) % 3] + i + 1) & MASK
        out.append((x0, x1))
    return out


def _key_schedule(seed_hi, seed_lo, n):
    k = (seed_hi, seed_lo)
    rows = []
    for _ in range(n):
        (a0, a1), (b0, b1), (c0, c1) = _tf_py(k[0], k[1], [(0, 0), (0, 1), (0, 2)])
        k = (a0, a1)
        rows.append((b0, b1, c0, c1))
    return np.asarray(rows, dtype=np.uint32)


_KEYS = _key_schedule(0, 42, _NITER)


def _threefry(k0, k1, x0, x1):
    ks2 = k0 ^ k1 ^ jnp.uint32(0x1BD11BDA)
    ks = (k0, k1, ks2)
    x0 = x0 + ks[0]
    x1 = x1 + ks[1]
    for i, rot in enumerate(((13, 15, 26, 6), (17, 29, 16, 24),
                             (13, 15, 26, 6), (17, 29, 16, 24),
                             (13, 15, 26, 6))):
        for r in rot:
            x0 = x0 + x1
            x1 = (x1 << jnp.uint32(r)) | (x1 >> jnp.uint32(32 - r))
            x1 = x0 ^ x1
        x0 = x0 + ks[(i + 1) % 3]
        x1 = x1 + ks[(i + 2) % 3] + jnp.uint32(i + 1)
    return x0, x1


def _uniform(bits):
    fb = (bits >> jnp.uint32(9)) | jnp.uint32(0x3F800000)
    f = lax.bitcast_convert_type(fb, jnp.float32) - jnp.float32(1.0)
    return jnp.maximum(jnp.float32(0.0), f)


def _lgamma(x):
    one = jnp.float32(1.0)
    z = x - one
    t0 = z + jnp.float32(0.5)
    t75 = z + jnp.float32(7.5)
    tlog = jnp.log1p(z * jnp.float32(0.13333334)) + jnp.float32(2.01490307)
    y1 = (t0 - t75 / tlog) * tlog + jnp.float32(0.918938518)
    acc = jnp.float32(676.520386) / (z + one) + one
    acc = acc + jnp.float32(-1259.13916) / (z + jnp.float32(2))
    acc = acc + jnp.float32(771.323425) / (z + jnp.float32(3))
    acc = acc + jnp.float32(-176.615036) / (z + jnp.float32(4))
    acc = acc + jnp.float32(12.5073433) / (z + jnp.float32(5))
    acc = acc + jnp.float32(-0.138571098) / (z + jnp.float32(6))
    acc = acc + jnp.float32(9.98436917e-06) / (z + jnp.float32(7))
    acc = acc + jnp.float32(1.50563267e-07) / (z + jnp.float32(8))
    return y1 + jnp.log(acc)


def _draw(k0, k1, k2, k3, flat, lam, a, b, inv_alpha, v_r, log_lam):
    zero = jnp.zeros_like(flat)
    o0, o1 = _threefry(k0, k1, zero, flat)
    u = _uniform(o0 ^ o1) - jnp.float32(0.5)
    o0, o1 = _threefry(k2, k3, zero, flat)
    v = _uniform(o0 ^ o1)
    u_shifted = jnp.float32(0.5) - jnp.abs(u)
    kc = jnp.floor((jnp.float32(2) * a / u_shifted + b) * u + lam
                   + jnp.float32(0.43))
    s = jnp.log(v * inv_alpha / (a / (u_shifted * u_shifted) + b))
    t = -lam + kc * log_lam - _lgamma(kc + jnp.float32(1))
    accept1 = (u_shifted >= jnp.float32(0.07)) & (v <= v_r)
    reject = (kc < jnp.float32(0)) | ((u_shifted < jnp.float32(0.013))
                                      & (v > u_shifted))
    accept = accept1 | (~reject & (s <= t))
    return kc, accept


def _mcount_kernel(lam_ref, a_ref, b_ref, ia_ref, vr_ref, ll_ref, keys_ref,
                   m_ref, *, lanes_total):
    ch = 8
    nchunk = (_T1 + ch - 1) // ch
    j0 = pl.program_id(0) * _BN_A
    jmat = lax.broadcasted_iota(jnp.int32, (ch, _BN_A), 1) + j0
    trel = lax.broadcasted_iota(jnp.int32, (ch, _BN_A), 0)
    lam = lam_ref[...]
    a = a_ref[...]
    b = b_ref[...]
    inv_alpha = ia_ref[...]
    v_r = vr_ref[...]
    log_lam = ll_ref[...]
    one = jnp.float32(1.0)

    def chunk_body(c, m_acc):
        tmat = trel + c * ch
        flat = (tmat * lanes_total + jmat).astype(jnp.uint32)
        acc0 = jnp.where(tmat >= _T1, one, jnp.float32(0.0))

        def cond(cr):
            i, acc_f = cr
            return (i < _NITER) & jnp.any(acc_f == 0)

        def body(cr):
            i, acc_f = cr
            _, accept = _draw(keys_ref[i, 0], keys_ref[i, 1], keys_ref[i, 2],
                              keys_ref[i, 3], flat, lam, a, b, inv_alpha,
                              v_r, log_lam)
            return i + 1, jnp.where(accept, one, acc_f)

        n, _ = lax.while_loop(cond, body, (jnp.int32(0), acc0))
        return jnp.maximum(m_acc, n)

    m_ref[0, 0, 0] = lax.fori_loop(0, nchunk, chunk_body, jnp.int32(0))


def _sample_kernel(lam_ref, a_ref, b_ref, ia_ref, vr_ref, ll_ref, act_ref,
                   keys_ref, mblk_ref, out_ref, *, lanes_total, n_mblk):
    def mx(g, m):
        return jnp.maximum(m, mblk_ref[g, 0, 0])

    m_glob = lax.fori_loop(0, n_mblk, mx, jnp.int32(0))

    ch = 8
    j0 = pl.program_id(0) * _BN_B
    jmat = lax.broadcasted_iota(jnp.int32, (ch, _BN_B), 1) + j0
    trel = lax.broadcasted_iota(jnp.int32, (ch, _BN_B), 0)
    lam = lam_ref[...]
    a = a_ref[...]
    b = b_ref[...]
    inv_alpha = ia_ref[...]
    v_r = vr_ref[...]
    log_lam = ll_ref[...]
    active = act_ref[...] != 0
    out_ref[...] = jnp.zeros((_STEPS, _BN_B), jnp.bool_)
    srow = (lax.broadcasted_iota(jnp.int32, (_STEPS, _BN_B), 0)
            + 1).astype(jnp.float32)

    one = jnp.float32(1.0)
    zerov8 = jnp.zeros((ch, _BN_B), jnp.float32)

    def tcond(c):
        t, _, done_f = c
        return (t < _T1) & jnp.any(done_f == 0)

    def tbody(c):
        t, acc, done_f = c
        tmat = trel + t
        valid_row = tmat < _T1
        flat = (tmat * lanes_total + jmat).astype(jnp.uint32)
        found0 = jnp.maximum(jnp.broadcast_to(done_f, (ch, _BN_B)),
                             jnp.where(valid_row, jnp.float32(0.0), one))

        def icond(ci):
            i, found_f, _ = ci
            return (i >= 0) & jnp.any(found_f == 0)

        def ibody(ci):
            i, found_f, kk = ci
            kc, accept = _draw(keys_ref[i, 0], keys_ref[i, 1], keys_ref[i, 2],
                               keys_ref[i, 3], flat, lam, a, b, inv_alpha,
                               v_r, log_lam)
            kk = jnp.where(accept & (found_f == 0), kc, kk)
            found_f = jnp.where(accept, one, found_f)
            return i - 1, found_f, kk

        _, _, kk = lax.while_loop(icond, ibody, (m_glob - 1, found0, zerov8))
        res = jnp.where(active & (kk == jnp.float32(0)), jnp.float32(1), kk)
        res = jnp.where(valid_row, res, jnp.float32(0.0))
        run = acc
        eq = jnp.zeros((_STEPS, _BN_B), jnp.bool_)
        for r in range(ch):
            run = run + res[r:r + 1, :]
            eq = eq | (srow == run)
        out_ref[...] = out_ref[...] | eq
        done_f = jnp.where(run > jnp.float32(_STEPS), one, done_f)
        return t + ch, run, done_f

    lax.while_loop(tcond, tbody,
                   (jnp.int32(0), jnp.zeros((1, _BN_B), jnp.float32),
                    jnp.where(active, jnp.float32(0), one)))


def kernel(inputs):
    shape = inputs.shape
    lanes = int(np.prod(shape))
    freq = _MAXFREQ * inputs
    rate = jnp.nan_to_num(1.0 / freq, posinf=0.0, neginf=0.0) * (1000.0 / 1.0)
    lam = rate.reshape(1, lanes).astype(jnp.float32)
    use_knuth = jnp.isnan(lam) | (lam < 10)
    lam_rej = jnp.where(use_knuth, jnp.float32(1e5), lam)
    log_lam = lax.log(lam_rej)
    b = 0.931 + 2.53 * lax.sqrt(lam_rej)
    a = -0.059 + 0.02483 * b
    inv_alpha = 1.1239 + 1.1328 / (b - 3.4)
    v_r = 0.9277 - 3.6224 / (b - 2)
    active = (~use_knuth).astype(jnp.int32)
    keys = jnp.asarray(_KEYS)

    grid_a = lanes // _BN_A
    param_spec_a = pl.BlockSpec((1, _BN_A), lambda j: (0, j))
    mblk = pl.pallas_call(
        lambda *refs: _mcount_kernel(*refs, lanes_total=lanes),
        grid=(grid_a,),
        in_specs=[param_spec_a] * 6 + [pl.BlockSpec(memory_space=pltpu.SMEM)],
        out_specs=pl.BlockSpec((1, 1, 1), lambda j: (j, 0, 0),
                               memory_space=pltpu.SMEM),
        out_shape=jax.ShapeDtypeStruct((grid_a, 1, 1), jnp.int32),
        compiler_params=pltpu.CompilerParams(
            dimension_semantics=("parallel",)),
    )(lam_rej, a, b, inv_alpha, v_r, log_lam, keys)

    grid_b = lanes // _BN_B
    param_spec_b = pl.BlockSpec((1, _BN_B), lambda j: (0, j))
    out = pl.pallas_call(
        lambda *refs: _sample_kernel(*refs, lanes_total=lanes, n_mblk=grid_a),
        grid=(grid_b,),
        in_specs=[param_spec_b] * 7 + [pl.BlockSpec(memory_space=pltpu.SMEM),
                                       pl.BlockSpec(memory_space=pltpu.SMEM)],
        out_specs=pl.BlockSpec((_STEPS, _BN_B), lambda j: (0, j)),
        out_shape=jax.ShapeDtypeStruct((_STEPS, lanes), jnp.bool_),
        compiler_params=pltpu.CompilerParams(
            dimension_semantics=("parallel",)),
    )(lam_rej, a, b, inv_alpha, v_r, log_lam, active, keys, mblk)

    return out.reshape((_STEPS,) + shape)


# speedup vs baseline: 1.1408x; 1.1408x over previous
import numpy as np
import jax
import jax.numpy as jnp
from jax import lax
from jax.experimental import pallas as pl
from jax.experimental.pallas import tpu as pltpu

_STEPS = 200
_T1 = _STEPS + 1
_MAXFREQ = 100.0
_NITER = 24
_BN_A = 2048
_BN_B = 2048


def _tf_py(k0, k1, xs):
    MASK = 0xFFFFFFFF
    ks = [k0, k1, k0 ^ k1 ^ 0x1BD11BDA]
    rots = ((13, 15, 26, 6), (17, 29, 16, 24)) * 3
    out = []
    for x0, x1 in xs:
        x0 = (x0 + ks[0]) & MASK
        x1 = (x1 + ks[1]) & MASK
        for i in range(5):
            for r in rots[i]:
                x0 = (x0 + x1) & MASK
                x1 = ((x1 << r) | (x1 >> (32 - r))) & MASK
                x1 = x0 ^ x1
            x0 = (x0 + ks[(i + 1) % 3]) & MASK
            x1 = (x1 + ks[(i + 2) % 3] + i + 1) & MASK
        out.append((x0, x1))
    return out


def _key_schedule(seed_hi, seed_lo, n):
    k = (seed_hi, seed_lo)
    rows = []
    for _ in range(n):
        (a0, a1), (b0, b1), (c0, c1) = _tf_py(k[0], k[1], [(0, 0), (0, 1), (0, 2)])
        k = (a0, a1)
        rows.append((b0, b1, c0, c1))
    return np.asarray(rows, dtype=np.uint32)


_KEYS = _key_schedule(0, 42, _NITER)


def _threefry(k0, k1, x0, x1):
    ks2 = k0 ^ k1 ^ jnp.uint32(0x1BD11BDA)
    ks = (k0, k1, ks2)
    x0 = x0 + ks[0]
    x1 = x1 + ks[1]
    for i, rot in enumerate(((13, 15, 26, 6), (17, 29, 16, 24),
                             (13, 15, 26, 6), (17, 29, 16, 24),
                             (13, 15, 26, 6))):
        for r in rot:
            x0 = x0 + x1
            x1 = (x1 << jnp.uint32(r)) | (x1 >> jnp.uint32(32 - r))
            x1 = x0 ^ x1
        x0 = x0 + ks[(i + 1) % 3]
        x1 = x1 + ks[(i + 2) % 3] + jnp.uint32(i + 1)
    return x0, x1


def _uniform(bits):
    fb = (bits >> jnp.uint32(9)) | jnp.uint32(0x3F800000)
    f = lax.bitcast_convert_type(fb, jnp.float32) - jnp.float32(1.0)
    return jnp.maximum(jnp.float32(0.0), f)


def _lgamma(x):
    one = jnp.float32(1.0)
    z = x - one
    t0 = z + jnp.float32(0.5)
    t75 = z + jnp.float32(7.5)
    tlog = jnp.log1p(z * jnp.float32(0.13333334)) + jnp.float32(2.01490307)
    y1 = (t0 - t75 / tlog) * tlog + jnp.float32(0.918938518)
    acc = jnp.float32(676.520386) / (z + one) + one
    acc = acc + jnp.float32(-1259.13916) / (z + jnp.float32(2))
    acc = acc + jnp.float32(771.323425) / (z + jnp.float32(3))
    acc = acc + jnp.float32(-176.615036) / (z + jnp.float32(4))
    acc = acc + jnp.float32(12.5073433) / (z + jnp.float32(5))
    acc = acc + jnp.float32(-0.138571098) / (z + jnp.float32(6))
    acc = acc + jnp.float32(9.98436917e-06) / (z + jnp.float32(7))
    acc = acc + jnp.float32(1.50563267e-07) / (z + jnp.float32(8))
    return y1 + jnp.log(acc)


def _draw(k0, k1, k2, k3, flat, lam, a, b, inv_alpha, v_r, log_lam):
    zero = jnp.zeros_like(flat)
    o0, o1 = _threefry(k0, k1, zero, flat)
    u = _uniform(o0 ^ o1) - jnp.float32(0.5)
    o0, o1 = _threefry(k2, k3, zero, flat)
    v = _uniform(o0 ^ o1)
    u_shifted = jnp.float32(0.5) - jnp.abs(u)
    kc = jnp.floor((jnp.float32(2) * a / u_shifted + b) * u + lam
                   + jnp.float32(0.43))
    s = jnp.log(v * inv_alpha / (a / (u_shifted * u_shifted) + b))
    t = -lam + kc * log_lam - _lgamma(kc + jnp.float32(1))
    accept1 = (u_shifted >= jnp.float32(0.07)) & (v <= v_r)
    reject = (kc < jnp.float32(0)) | ((u_shifted < jnp.float32(0.013))
                                      & (v > u_shifted))
    accept = accept1 | (~reject & (s <= t))
    return kc, accept


def _mcount_kernel(lam_ref, a_ref, b_ref, ia_ref, vr_ref, ll_ref, keys_ref,
                   m_ref, *, lanes_total):
    ch = 8
    nchunk = (_T1 + ch - 1) // ch
    j0 = pl.program_id(0) * _BN_A
    jmat = lax.broadcasted_iota(jnp.int32, (ch, _BN_A), 1) + j0
    trel = lax.broadcasted_iota(jnp.int32, (ch, _BN_A), 0)
    zero8 = jnp.zeros((ch, _BN_A), jnp.float32)
    lam = lam_ref[...] + zero8
    a = a_ref[...] + zero8
    b = b_ref[...] + zero8
    inv_alpha = ia_ref[...] + zero8
    v_r = vr_ref[...] + zero8
    log_lam = ll_ref[...] + zero8
    one = jnp.float32(1.0)

    def chunk_body(c, m_acc):
        tmat = trel + c * ch
        flat = (tmat * lanes_total + jmat).astype(jnp.uint32)
        acc0 = jnp.where(tmat >= _T1, one, jnp.float32(0.0))

        def cond(cr):
            i, acc_f = cr
            return (i < _NITER) & jnp.any(acc_f == 0)

        def body(cr):
            i, acc_f = cr
            _, accept = _draw(keys_ref[i, 0], keys_ref[i, 1], keys_ref[i, 2],
                              keys_ref[i, 3], flat, lam, a, b, inv_alpha,
                              v_r, log_lam)
            return i + 1, jnp.where(accept, one, acc_f)

        n, _ = lax.while_loop(cond, body, (jnp.int32(0), acc0))
        return jnp.maximum(m_acc, n)

    m_ref[0, 0, 0] = lax.fori_loop(0, nchunk, chunk_body, jnp.int32(0))


def _sample_kernel(lam_ref, a_ref, b_ref, ia_ref, vr_ref, ll_ref, act_ref,
                   keys_ref, mblk_ref, out_ref, *, lanes_total, n_mblk):
    def mx(g, m):
        return jnp.maximum(m, mblk_ref[g, 0, 0])

    m_glob = lax.fori_loop(0, n_mblk, mx, jnp.int32(0))

    ch = 8
    j0 = pl.program_id(0) * _BN_B
    jmat = lax.broadcasted_iota(jnp.int32, (ch, _BN_B), 1) + j0
    trel = lax.broadcasted_iota(jnp.int32, (ch, _BN_B), 0)
    zero8 = jnp.zeros((ch, _BN_B), jnp.float32)
    lam = lam_ref[...] + zero8
    a = a_ref[...] + zero8
    b = b_ref[...] + zero8
    inv_alpha = ia_ref[...] + zero8
    v_r = vr_ref[...] + zero8
    log_lam = ll_ref[...] + zero8
    active = act_ref[...] != 0
    out_ref[...] = jnp.zeros((_STEPS, _BN_B), jnp.bool_)
    srow = (lax.broadcasted_iota(jnp.int32, (_STEPS, _BN_B), 0)
            + 1).astype(jnp.float32)

    one = jnp.float32(1.0)
    zerov8 = jnp.zeros((ch, _BN_B), jnp.float32)

    def tcond(c):
        t, _, done_f = c
        return (t < _T1) & jnp.any(done_f == 0)

    def tbody(c):
        t, acc, done_f = c
        tmat = trel + t
        valid_row = tmat < _T1
        flat = (tmat * lanes_total + jmat).astype(jnp.uint32)
        found0 = jnp.maximum(jnp.broadcast_to(done_f, (ch, _BN_B)),
                             jnp.where(valid_row, jnp.float32(0.0), one))

        def icond(ci):
            i, found_f, _ = ci
            return (i >= 0) & jnp.any(found_f == 0)

        def ibody(ci):
            i, found_f, kk = ci
            kc, accept = _draw(keys_ref[i, 0], keys_ref[i, 1], keys_ref[i, 2],
                               keys_ref[i, 3], flat, lam, a, b, inv_alpha,
                               v_r, log_lam)
            kk = jnp.where(accept & (found_f == 0), kc, kk)
            found_f = jnp.where(accept, one, found_f)
            return i - 1, found_f, kk

        _, _, kk = lax.while_loop(icond, ibody, (m_glob - 1, found0, zerov8))
        res = jnp.where(active & (kk == jnp.float32(0)), jnp.float32(1), kk)
        res = jnp.where(valid_row, res, jnp.float32(0.0))
        run = acc
        eq = jnp.zeros((_STEPS, _BN_B), jnp.bool_)
        for r in range(ch):
            run = run + res[r:r + 1, :]
            eq = eq | (srow == run)
        out_ref[...] = out_ref[...] | eq
        done_f = jnp.where(run > jnp.float32(_STEPS), one, done_f)
        return t + ch, run, done_f

    lax.while_loop(tcond, tbody,
                   (jnp.int32(0), jnp.zeros((1, _BN_B), jnp.float32),
                    jnp.where(active, jnp.float32(0), one)))


def kernel(inputs):
    shape = inputs.shape
    lanes = int(np.prod(shape))
    freq = _MAXFREQ * inputs
    rate = jnp.nan_to_num(1.0 / freq, posinf=0.0, neginf=0.0) * (1000.0 / 1.0)
    lam = rate.reshape(1, lanes).astype(jnp.float32)
    use_knuth = jnp.isnan(lam) | (lam < 10)
    lam_rej = jnp.where(use_knuth, jnp.float32(1e5), lam)
    log_lam = lax.log(lam_rej)
    b = 0.931 + 2.53 * lax.sqrt(lam_rej)
    a = -0.059 + 0.02483 * b
    inv_alpha = 1.1239 + 1.1328 / (b - 3.4)
    v_r = 0.9277 - 3.6224 / (b - 2)
    active = (~use_knuth).astype(jnp.int32)
    keys = jnp.asarray(_KEYS)

    grid_a = lanes // _BN_A
    param_spec_a = pl.BlockSpec((1, _BN_A), lambda j: (0, j))
    mblk = pl.pallas_call(
        lambda *refs: _mcount_kernel(*refs, lanes_total=lanes),
        grid=(grid_a,),
        in_specs=[param_spec_a] * 6 + [pl.BlockSpec(memory_space=pltpu.SMEM)],
        out_specs=pl.BlockSpec((1, 1, 1), lambda j: (j, 0, 0),
                               memory_space=pltpu.SMEM),
        out_shape=jax.ShapeDtypeStruct((grid_a, 1, 1), jnp.int32),
        compiler_params=pltpu.CompilerParams(
            dimension_semantics=("parallel",)),
    )(lam_rej, a, b, inv_alpha, v_r, log_lam, keys)

    grid_b = lanes // _BN_B
    param_spec_b = pl.BlockSpec((1, _BN_B), lambda j: (0, j))
    out = pl.pallas_call(
        lambda *refs: _sample_kernel(*refs, lanes_total=lanes, n_mblk=grid_a),
        grid=(grid_b,),
        in_specs=[param_spec_b] * 7 + [pl.BlockSpec(memory_space=pltpu.SMEM),
                                       pl.BlockSpec(memory_space=pltpu.SMEM)],
        out_specs=pl.BlockSpec((_STEPS, _BN_B), lambda j: (0, j)),
        out_shape=jax.ShapeDtypeStruct((_STEPS, lanes), jnp.bool_),
        compiler_params=pltpu.CompilerParams(
            dimension_semantics=("parallel",)),
    )(lam_rej, a, b, inv_alpha, v_r, log_lam, active, keys, mblk)

    return out.reshape((_STEPS,) + shape)
